# baseline (speedup 1.0000x reference)
import functools

import jax
import jax.numpy as jnp
import numpy as np
from jax import lax
from jax.experimental import pallas as pl
from jax.experimental.pallas import tpu as pltpu
from jax.experimental.pallas import tpu_sc as plsc

N = 10000
D = 128
E = 320000
EPS = 1e-5

N_PAD = 10240
CHUNK = 128
GROUP = 8
NUM_CORES = 2
NUM_TILES = 16
NUM_WORKERS = NUM_CORES * NUM_TILES
CHUNKS_PER_TILE = 80
NUM_CHUNKS = NUM_WORKERS * CHUNKS_PER_TILE
E_PAD = NUM_CHUNKS * CHUNK
NUM_GROUPS = CHUNKS_PER_TILE // GROUP
ROWS_PER_TILE = N_PAD // NUM_TILES
REAL_ALIGNED = (E // CHUNK) // GROUP * GROUP
PAD_ROWS = NUM_CHUNKS - REAL_ALIGNED

_r = np.arange(E_PAD - E)
_PAD_NP = np.stack([_r % N, N + _r % (N_PAD - N)]).astype(np.int32)

_mesh = plsc.VectorSubcoreMesh(core_axis_name="c", subcore_axis_name="s")


@functools.partial(
    pl.kernel,
    out_type=(
        jax.ShapeDtypeStruct((NUM_CORES, N_PAD, D), jnp.float32),
        jax.ShapeDtypeStruct((NUM_CORES, N_PAD), jnp.float32),
    ),
    mesh=_mesh,
    compiler_params=pltpu.CompilerParams(use_tc_tiling_on_sc=False),
    scratch_types=[
        pltpu.VMEM((2, GROUP, CHUNK), jnp.int32),
        pltpu.VMEM((2, GROUP, CHUNK), jnp.int32),
        pltpu.VMEM((2, CHUNK, D), jnp.float32),
        pltpu.VMEM((CHUNK,), jnp.float32),
        pltpu.VMEM((CHUNK,), jnp.float32),
        pltpu.VMEM_SHARED((N_PAD, D), jnp.float32),
        pltpu.VMEM_SHARED((N_PAD,), jnp.float32),
        pltpu.SemaphoreType.DMA,
        pltpu.SemaphoreType.DMA,
        pltpu.SemaphoreType.DMA,
    ],
)
def _sc_aggregate(feature_hbm, edge_hbm, pad_hbm, sum_hbm, deg_hbm,
                  src_v, dst_v, rows_v, ones_v, zeros_v,
                  acc_sh, deg_sh, gsem, dsem, isem):
    c = lax.axis_index("c")
    s = lax.axis_index("s")
    rows = (rows_v.at[0], rows_v.at[1])
    rows_a = rows[0]

    def zero_rows(r, carry):
        for k in range(D // 16):
            rows_a[r, pl.ds(k * 16, 16)] = jnp.zeros((16,), jnp.float32)
        return carry
    lax.fori_loop(0, CHUNK, zero_rows, 0)

    def init_small(r, carry):
        ones_v[pl.ds(r * 16, 16)] = jnp.ones((16,), jnp.float32)
        zeros_v[pl.ds(r * 16, 16)] = jnp.zeros((16,), jnp.float32)
        return carry
    lax.fori_loop(0, CHUNK // 16, init_small, 0)

    base = s * ROWS_PER_TILE
    zhs = []
    for j in range(ROWS_PER_TILE // CHUNK):
        zhs.append(pltpu.async_copy(
            rows_a, acc_sh.at[pl.ds(base + j * CHUNK, CHUNK)], dsem))
        zhs.append(pltpu.async_copy(
            zeros_v, deg_sh.at[pl.ds(base + j * CHUNK, CHUNK)], dsem))
    for h in zhs:
        h.wait()
    plsc.subcore_barrier()

    start_chunk = (c * NUM_TILES + s) * CHUNKS_PER_TILE

    def fire_idx(g, b):
        row0 = start_chunk + (g % NUM_GROUPS) * GROUP

        @pl.when(row0 < REAL_ALIGNED)
        def _():
            pltpu.async_copy(edge_hbm.at[0, pl.ds(row0, GROUP)],
                             src_v.at[b], isem)
            pltpu.async_copy(edge_hbm.at[1, pl.ds(row0, GROUP)],
                             dst_v.at[b], isem)

        @pl.when(row0 >= REAL_ALIGNED)
        def _():
            prow = row0 - REAL_ALIGNED
            pltpu.async_copy(pad_hbm.at[0, pl.ds(prow, GROUP)],
                             src_v.at[b], isem)
            pltpu.async_copy(pad_hbm.at[1, pl.ds(prow, GROUP)],
                             dst_v.at[b], isem)

    def wait_idx():
        pltpu.make_async_copy(edge_hbm.at[0, pl.ds(0, GROUP)], src_v.at[0],
                              isem).wait()
        pltpu.make_async_copy(edge_hbm.at[0, pl.ds(0, GROUP)], dst_v.at[0],
                              isem).wait()

    def fire_gather(b, j, u):
        return pltpu.async_copy(feature_hbm.at[src_v.at[b, j]], rows[u], gsem)

    def drain_gather():
        pltpu.make_async_copy(feature_hbm.at[pl.ds(0, CHUNK)], rows[0],
                              gsem).wait()

    def consume(b, j, u):
        drain_gather()
        pltpu.sync_copy(rows[u], acc_sh.at[dst_v.at[b, j]], add=True)
        pltpu.async_copy(ones_v, deg_sh.at[dst_v.at[b, j]], dsem, add=True)

    def drain_deg():
        pltpu.make_async_copy(feature_hbm.at[0], zeros_v, dsem).wait()

    fire_idx(0, 0)
    wait_idx()
    fire_gather(0, 0, 0)
    fire_gather(0, 1, 1)

    def group_body(g, carry):
        b = g % 2
        bn = (g + 1) % 2

        @pl.when(g > 0)
        def _():
            for _u in range(GROUP):
                drain_deg()

        fire_idx(g + 1, bn)
        for j in range(GROUP):
            consume(b, j, j % 2)
            if j < GROUP - 2:
                fire_gather(b, j + 2, j % 2)
            elif j == GROUP - 2:
                wait_idx()
                fire_gather(bn, 0, j % 2)
            else:
                fire_gather(bn, 1, j % 2)
        return carry
    lax.fori_loop(0, NUM_GROUPS, group_body, 0)

    drain_gather()
    drain_gather()
    for _u in range(GROUP):
        drain_deg()
    plsc.subcore_barrier()

    pltpu.sync_copy(acc_sh.at[pl.ds(base, ROWS_PER_TILE)],
                    sum_hbm.at[c, pl.ds(base, ROWS_PER_TILE)])
    pltpu.sync_copy(deg_sh.at[pl.ds(base, ROWS_PER_TILE)],
                    deg_hbm.at[c, pl.ds(base, ROWS_PER_TILE)])


def _tc_body(psum_ref, pdeg_ref, feat_ref, w_ref, b_ref, gamma_ref, beta_ref,
             out_ref):
    ssum = psum_ref[0] + psum_ref[1]
    deg = pdeg_ref[0] + pdeg_ref[1]
    h = ssum[:N] / jnp.maximum(deg[:N], 1.0).reshape(N, 1)
    h = lax.dot_general(h, w_ref[...], (((1,), (1,)), ((), ())),
                        preferred_element_type=jnp.float32)
    h = h + b_ref[...]
    mean = jnp.mean(h, axis=0, keepdims=True)
    var = jnp.mean((h - mean) ** 2, axis=0, keepdims=True)
    h = (h - mean) * (lax.rsqrt(var + EPS) * gamma_ref[...]) + beta_ref[...]
    out_ref[...] = feat_ref[...] + jnp.maximum(h, 0.0)


_tc_update = pl.pallas_call(
    _tc_body,
    out_shape=jax.ShapeDtypeStruct((N, D), jnp.float32),
)


def kernel(feature, edge_index, W, b, gamma, beta):
    edge3d = edge_index.reshape(2, E // CHUNK, CHUNK)
    pad64 = jnp.concatenate(
        [edge_index[:, REAL_ALIGNED * CHUNK:], jnp.asarray(_PAD_NP)],
        axis=1).reshape(2, PAD_ROWS, CHUNK)
    psum, pdeg = _sc_aggregate(feature, edge3d, pad64)
    return _tc_update(psum, pdeg, feature, W,
                      b.reshape(1, D), gamma.reshape(1, D), beta.reshape(1, D))


# speedup vs baseline: 1.0384x; 1.0384x over previous
import functools

import jax
import jax.numpy as jnp
import numpy as np
from jax import lax
from jax.experimental import pallas as pl
from jax.experimental.pallas import tpu as pltpu
from jax.experimental.pallas import tpu_sc as plsc

N = 10000
D = 128
E = 320000
EPS = 1e-5

N_PAD = 10240
CHUNK = 128
GROUP = 8
NUM_CORES = 2
NUM_TILES = 16
NUM_WORKERS = NUM_CORES * NUM_TILES
CHUNKS_PER_TILE = 80
NUM_CHUNKS = NUM_WORKERS * CHUNKS_PER_TILE
E_PAD = NUM_CHUNKS * CHUNK
NUM_GROUPS = CHUNKS_PER_TILE // GROUP
ROWS_PER_TILE = N_PAD // NUM_TILES
REAL_ALIGNED = (E // CHUNK) // GROUP * GROUP
PAD_ROWS = NUM_CHUNKS - REAL_ALIGNED

_r = np.arange(E_PAD - E)
_PAD_NP = np.stack([_r % N, N + _r % (N_PAD - N)]).astype(np.int32)

_mesh = plsc.VectorSubcoreMesh(core_axis_name="c", subcore_axis_name="s")


@functools.partial(
    pl.kernel,
    out_type=(
        jax.ShapeDtypeStruct((NUM_CORES, N_PAD, D), jnp.float32),
        jax.ShapeDtypeStruct((NUM_CORES, N_PAD), jnp.float32),
    ),
    mesh=_mesh,
    scratch_types=[
        pltpu.VMEM((2, GROUP * CHUNK), jnp.int32),
        pltpu.VMEM((2, GROUP, CHUNK), jnp.int32),
        pltpu.VMEM((2, CHUNK, D), jnp.float32),
        pltpu.VMEM((CHUNK,), jnp.float32),
        pltpu.VMEM((CHUNK,), jnp.float32),
        pltpu.VMEM_SHARED((N_PAD, D), jnp.float32),
        pltpu.VMEM_SHARED((N_PAD,), jnp.float32),
        pltpu.SemaphoreType.DMA,
        pltpu.SemaphoreType.DMA,
        pltpu.SemaphoreType.DMA,
    ],
)
def _sc_aggregate(feature_hbm, edge_hbm, pad_hbm, sum_hbm, deg_hbm,
                  src_v, dst_v, rows_v, ones_v, zeros_v,
                  acc_sh, deg_sh, gsem, dsem, isem):
    c = lax.axis_index("c")
    s = lax.axis_index("s")
    rows = (rows_v.at[0], rows_v.at[1])
    rows_a = rows[0]

    def zero_rows(r, carry):
        for k in range(D // 16):
            rows_a[r, pl.ds(k * 16, 16)] = jnp.zeros((16,), jnp.float32)
        return carry
    lax.fori_loop(0, CHUNK, zero_rows, 0)

    def init_small(r, carry):
        ones_v[pl.ds(r * 16, 16)] = jnp.ones((16,), jnp.float32)
        zeros_v[pl.ds(r * 16, 16)] = jnp.zeros((16,), jnp.float32)
        return carry
    lax.fori_loop(0, CHUNK // 16, init_small, 0)

    base = s * ROWS_PER_TILE
    zhs = []
    for j in range(ROWS_PER_TILE // CHUNK):
        zhs.append(pltpu.async_copy(
            rows_a, acc_sh.at[pl.ds(base + j * CHUNK, CHUNK)], dsem))
        zhs.append(pltpu.async_copy(
            zeros_v, deg_sh.at[pl.ds(base + j * CHUNK, CHUNK)], dsem))
    for h in zhs:
        h.wait()
    plsc.subcore_barrier()

    start_chunk = (c * NUM_TILES + s) * CHUNKS_PER_TILE

    def fire_idx(g, b):
        row0 = start_chunk + (g % NUM_GROUPS) * GROUP
        e0 = row0 * CHUNK

        @pl.when(row0 < REAL_ALIGNED)
        def _():
            pltpu.async_copy(edge_hbm.at[0, pl.ds(e0, GROUP * CHUNK)],
                             src_v.at[b], isem)
            for j in range(GROUP):
                pltpu.async_copy(edge_hbm.at[1, pl.ds(e0 + j * CHUNK, CHUNK)],
                                 dst_v.at[b, j], isem)

        @pl.when(row0 >= REAL_ALIGNED)
        def _():
            p0 = e0 - REAL_ALIGNED * CHUNK
            pltpu.async_copy(pad_hbm.at[0, pl.ds(p0, GROUP * CHUNK)],
                             src_v.at[b], isem)
            for j in range(GROUP):
                pltpu.async_copy(pad_hbm.at[1, pl.ds(p0 + j * CHUNK, CHUNK)],
                                 dst_v.at[b, j], isem)

    def wait_idx():
        pltpu.make_async_copy(edge_hbm.at[0, pl.ds(0, GROUP * CHUNK)],
                              src_v.at[0], isem).wait()
        pltpu.make_async_copy(edge_hbm.at[0, pl.ds(0, GROUP * CHUNK)],
                              src_v.at[1], isem).wait()

    def fire_gather(b, j, u):
        return pltpu.async_copy(
            feature_hbm.at[src_v.at[b, pl.ds(j * CHUNK, CHUNK)]], rows[u],
            gsem)

    def drain_gather():
        pltpu.make_async_copy(feature_hbm.at[pl.ds(0, CHUNK)], rows[0],
                              gsem).wait()

    def consume(b, j, u):
        drain_gather()
        pltpu.sync_copy(rows[u], acc_sh.at[dst_v.at[b, j]], add=True)
        pltpu.async_copy(ones_v, deg_sh.at[dst_v.at[b, j]], dsem, add=True)

    def drain_deg():
        pltpu.make_async_copy(feature_hbm.at[0], zeros_v, dsem).wait()

    fire_idx(0, 0)
    wait_idx()
    fire_gather(0, 0, 0)
    fire_gather(0, 1, 1)

    def group_body(g, carry):
        b = g % 2
        bn = (g + 1) % 2

        @pl.when(g > 0)
        def _():
            for _u in range(GROUP):
                drain_deg()

        fire_idx(g + 1, bn)
        for j in range(GROUP):
            consume(b, j, j % 2)
            if j < GROUP - 2:
                fire_gather(b, j + 2, j % 2)
            elif j == GROUP - 2:
                wait_idx()
                fire_gather(bn, 0, j % 2)
            else:
                fire_gather(bn, 1, j % 2)
        return carry
    lax.fori_loop(0, NUM_GROUPS, group_body, 0)

    drain_gather()
    drain_gather()
    for _u in range(GROUP):
        drain_deg()
    plsc.subcore_barrier()

    pltpu.sync_copy(acc_sh.at[pl.ds(base, ROWS_PER_TILE)],
                    sum_hbm.at[c, pl.ds(base, ROWS_PER_TILE)])
    pltpu.sync_copy(deg_sh.at[pl.ds(base, ROWS_PER_TILE)],
                    deg_hbm.at[c, pl.ds(base, ROWS_PER_TILE)])


def _tc_body(psum_ref, pdeg_ref, feat_ref, w_ref, b_ref, gamma_ref, beta_ref,
             out_ref):
    ssum = psum_ref[0] + psum_ref[1]
    deg = pdeg_ref[0] + pdeg_ref[1]
    h = ssum[:N] / jnp.maximum(deg[:N], 1.0).reshape(N, 1)
    h = lax.dot_general(h, w_ref[...], (((1,), (1,)), ((), ())),
                        preferred_element_type=jnp.float32)
    h = h + b_ref[...]
    mean = jnp.mean(h, axis=0, keepdims=True)
    var = jnp.mean((h - mean) ** 2, axis=0, keepdims=True)
    h = (h - mean) * (lax.rsqrt(var + EPS) * gamma_ref[...]) + beta_ref[...]
    out_ref[...] = feat_ref[...] + jnp.maximum(h, 0.0)


_tc_update = pl.pallas_call(
    _tc_body,
    out_shape=jax.ShapeDtypeStruct((N, D), jnp.float32),
)


def kernel(feature, edge_index, W, b, gamma, beta):
    pad64 = jnp.concatenate(
        [edge_index[:, REAL_ALIGNED * CHUNK:], jnp.asarray(_PAD_NP)], axis=1)
    psum, pdeg = _sc_aggregate(feature, edge_index, pad64)
    return _tc_update(psum, pdeg, feature, W,
                      b.reshape(1, D), gamma.reshape(1, D), beta.reshape(1, D))
